# baseline (speedup 1.0000x reference)
import functools

import jax
import jax.numpy as jnp
from jax import lax
from jax.experimental import pallas as pl
from jax.experimental.pallas import tpu as pltpu
from jax.experimental.pallas import tpu_sc as plsc

_CHUNK = 128


@functools.lru_cache(maxsize=None)
def _make(b_flat: int, d: int):
    info = plsc.get_sparse_core_info()
    nc, ns = info.num_cores, info.num_subcores
    nw = nc * ns
    n_chunks = b_flat // _CHUNK
    per_w = n_chunks // nw
    mesh = plsc.VectorSubcoreMesh(core_axis_name="c", subcore_axis_name="s")

    @functools.partial(
        pl.kernel,
        mesh=mesh,
        out_type=jax.ShapeDtypeStruct((b_flat, d), jnp.float32),
        scratch_types=[
            pltpu.VMEM((per_w, _CHUNK), jnp.int32),
            pltpu.VMEM((_CHUNK, d), jnp.float32),
            pltpu.SemaphoreType.DMA,
        ],
        compiler_params=pltpu.CompilerParams(use_tc_tiling_on_sc=False),
    )
    def gather_kernel(table_hbm, idx_hbm, out_hbm, idx_v, rows_v, sem):
        wid = lax.axis_index("s") * nc + lax.axis_index("c")
        c0 = wid * per_w
        pltpu.sync_copy(idx_hbm.at[pl.ds(c0, per_w)], idx_v)

        def body(j, carry):
            pltpu.async_copy(table_hbm.at[idx_v.at[j]], rows_v, sem).wait()
            pltpu.sync_copy(rows_v, out_hbm.at[pl.ds((c0 + j) * _CHUNK, _CHUNK)])
            return carry

        lax.fori_loop(0, per_w, body, 0)

    return gather_kernel


def kernel(x, weight):
    b, s = x.shape
    d = weight.shape[1]
    b_flat = b * s
    xf = x.reshape(b_flat // _CHUNK, _CHUNK).astype(jnp.int32)
    out = _make(b_flat, d)(weight, xf)
    return out.reshape(b, s, d)


# speedup vs baseline: 1.0732x; 1.0732x over previous
import functools

import jax
import jax.numpy as jnp
from jax import lax
from jax.experimental import pallas as pl
from jax.experimental.pallas import tpu as pltpu
from jax.experimental.pallas import tpu_sc as plsc

_CHUNK = 128
_K = 4


@functools.lru_cache(maxsize=None)
def _make(b_flat: int, d: int):
    info = plsc.get_sparse_core_info()
    nc, ns = info.num_cores, info.num_subcores
    nw = nc * ns
    n_chunks = b_flat // _CHUNK
    per_w = n_chunks // nw
    n_rounds = per_w // _K
    n_super = n_rounds // 2
    mesh = plsc.VectorSubcoreMesh(core_axis_name="c", subcore_axis_name="s")

    @functools.partial(
        pl.kernel,
        mesh=mesh,
        out_type=jax.ShapeDtypeStruct((b_flat, d), jnp.float32),
        scratch_types=[
            pltpu.VMEM((per_w, _CHUNK), jnp.int32),
            pltpu.VMEM((2, _K, _CHUNK, d), jnp.float32),
            pltpu.SemaphoreType.DMA,
            pltpu.SemaphoreType.DMA,
            pltpu.SemaphoreType.DMA,
            pltpu.SemaphoreType.DMA,
        ],
        compiler_params=pltpu.CompilerParams(use_tc_tiling_on_sc=False),
    )
    def gather_kernel(table_hbm, idx_hbm, out_hbm, idx_v, rows_v,
                      sem_g0, sem_g1, sem_w0, sem_w1):
        wid = lax.axis_index("s") * nc + lax.axis_index("c")
        c0 = wid * per_w
        pltpu.sync_copy(idx_hbm.at[pl.ds(c0, per_w)], idx_v)

        sem_g = (sem_g0, sem_g1)
        sem_w = (sem_w0, sem_w1)

        def fire_gathers(r, g):
            for b in range(_K):
                pltpu.async_copy(
                    table_hbm.at[idx_v.at[r * _K + b]],
                    rows_v.at[g, b], sem_g[g])

        def drain_gathers(g):
            for b in range(_K):
                pltpu.make_async_copy(
                    table_hbm.at[pl.ds(0, _CHUNK)],
                    rows_v.at[g, b], sem_g[g]).wait()

        def fire_writes(r, g):
            for b in range(_K):
                pltpu.async_copy(
                    rows_v.at[g, b],
                    out_hbm.at[pl.ds((c0 + r * _K + b) * _CHUNK, _CHUNK)],
                    sem_w[g])

        def drain_writes(g):
            for b in range(_K):
                pltpu.make_async_copy(
                    rows_v.at[g, b],
                    out_hbm.at[pl.ds(0, _CHUNK)], sem_w[g]).wait()

        fire_gathers(0, 0)

        def body(t, carry):
            r0 = 2 * t
            drain_gathers(0)
            fire_writes(r0, 0)

            @pl.when(t > 0)
            def _():
                drain_writes(1)
            fire_gathers(r0 + 1, 1)

            drain_gathers(1)
            fire_writes(r0 + 1, 1)
            drain_writes(0)

            @pl.when(t < n_super - 1)
            def _():
                fire_gathers(r0 + 2, 0)
            return carry

        lax.fori_loop(0, n_super, body, 0)
        drain_writes(1)

    return gather_kernel


def kernel(x, weight):
    b, s = x.shape
    d = weight.shape[1]
    b_flat = b * s
    xf = x.reshape(b_flat // _CHUNK, _CHUNK).astype(jnp.int32)
    out = _make(b_flat, d)(weight, xf)
    return out.reshape(b, s, d)
